# Initial kernel scaffold; baseline (speedup 1.0000x reference)
import jax
import jax.numpy as jnp
from jax.experimental import pallas as pl


def kernel(step, gamma):
    raise NotImplementedError("write your pallas kernel here")


# speedup vs baseline: 6.1964x; 6.1964x over previous
import functools

import jax
import jax.numpy as jnp
from jax import lax
from jax.experimental import pallas as pl
from jax.experimental.pallas import tpu as pltpu
from jax.experimental.pallas import tpu_sc as plsc

_TIMESTEPS = 100
_BATCH = 16384
_NW = 32
_PER_W = _BATCH // _NW
_LANES = 16
_TABLE = _TIMESTEPS + 1
_MAGIC = 12582912.0


def _make_kernel():
    mesh = plsc.VectorSubcoreMesh(core_axis_name="c", subcore_axis_name="s")

    @functools.partial(
        pl.kernel,
        mesh=mesh,
        out_type=jax.ShapeDtypeStruct((_BATCH,), jnp.float32),
        compiler_params=pltpu.CompilerParams(needs_layout_passes=False),
        scratch_types=[
            pltpu.VMEM((_PER_W,), jnp.float32),
            pltpu.VMEM((_TABLE,), jnp.float32),
            pltpu.VMEM((_PER_W,), jnp.float32),
        ],
    )
    def _noise_lookup(step_hbm, gamma_hbm, out_hbm, step_v, gamma_v, out_v):
        wid = lax.axis_index("s") * 2 + lax.axis_index("c")
        base = wid * _PER_W
        pltpu.sync_copy(step_hbm.at[pl.ds(base, _PER_W)], step_v)
        pltpu.sync_copy(gamma_hbm, gamma_v)
        for i in range(_PER_W // _LANES):
            x = step_v[pl.ds(i * _LANES, _LANES)]
            r = (x * float(_TIMESTEPS) + _MAGIC) - _MAGIC
            idx = r.astype(jnp.int32)
            out_v[pl.ds(i * _LANES, _LANES)] = plsc.load_gather(gamma_v, [idx])
        pltpu.sync_copy(out_v, out_hbm.at[pl.ds(base, _PER_W)])

    return _noise_lookup


_kernel_fn = _make_kernel()


def kernel(step, gamma):
    return _kernel_fn(step, gamma)


# speedup vs baseline: 6.3616x; 1.0267x over previous
import functools

import jax
import jax.numpy as jnp
from jax import lax
from jax.experimental import pallas as pl
from jax.experimental.pallas import tpu as pltpu
from jax.experimental.pallas import tpu_sc as plsc

_TIMESTEPS = 100
_BATCH = 16384
_NW = 32
_PER_W = _BATCH // _NW
_LANES = 16
_TABLE = _TIMESTEPS + 1
_MAGIC = 12582912.0


def _make_kernel():
    mesh = plsc.VectorSubcoreMesh(core_axis_name="c", subcore_axis_name="s")

    @functools.partial(
        pl.kernel,
        mesh=mesh,
        out_type=jax.ShapeDtypeStruct((_BATCH,), jnp.float32),
        compiler_params=pltpu.CompilerParams(needs_layout_passes=False),
        scratch_types=[
            pltpu.VMEM((_PER_W,), jnp.float32),
            pltpu.VMEM((_TABLE,), jnp.float32),
            pltpu.VMEM((_PER_W,), jnp.float32),
        ],
    )
    def _noise_lookup(step_hbm, gamma_hbm, out_hbm, step_v, gamma_v, out_v):
        wid = lax.axis_index("s") * 2 + lax.axis_index("c")
        base = wid * _PER_W
        pltpu.sync_copy(step_hbm.at[pl.ds(base, _PER_W)], step_v)
        pltpu.sync_copy(gamma_hbm, gamma_v)
        @plsc.parallel_loop(0, _PER_W, step=_LANES)
        def _body(i):
            off = pl.multiple_of(i, _LANES)
            x = step_v[pl.ds(off, _LANES)]
            r = (x * float(_TIMESTEPS) + _MAGIC) - _MAGIC
            idx = r.astype(jnp.int32)
            out_v[pl.ds(off, _LANES)] = plsc.load_gather(gamma_v, [idx])
        pltpu.sync_copy(out_v, out_hbm.at[pl.ds(base, _PER_W)])

    return _noise_lookup


_kernel_fn = _make_kernel()


def kernel(step, gamma):
    return _kernel_fn(step, gamma)


# speedup vs baseline: 6.8268x; 1.0731x over previous
import functools

import jax
import jax.numpy as jnp
from jax import lax
from jax.experimental import pallas as pl
from jax.experimental.pallas import tpu as pltpu
from jax.experimental.pallas import tpu_sc as plsc

_TIMESTEPS = 100
_BATCH = 16384
_NC = 1
_NW = 16 * _NC
_PER_W = _BATCH // _NW
_LANES = 16
_TABLE = _TIMESTEPS + 1
_MAGIC = 12582912.0


def _make_kernel():
    mesh = plsc.VectorSubcoreMesh(
        core_axis_name="c", subcore_axis_name="s", num_cores=_NC)

    @functools.partial(
        pl.kernel,
        mesh=mesh,
        out_type=jax.ShapeDtypeStruct((_BATCH,), jnp.float32),
        compiler_params=pltpu.CompilerParams(needs_layout_passes=False),
        scratch_types=[
            pltpu.VMEM((_PER_W,), jnp.float32),
            pltpu.VMEM((_TABLE,), jnp.float32),
            pltpu.VMEM((_PER_W,), jnp.float32),
        ],
    )
    def _noise_lookup(step_hbm, gamma_hbm, out_hbm, step_v, gamma_v, out_v):
        wid = lax.axis_index("s") * _NC + lax.axis_index("c")
        base = wid * _PER_W
        pltpu.sync_copy(step_hbm.at[pl.ds(base, _PER_W)], step_v)
        pltpu.sync_copy(gamma_hbm, gamma_v)
        @plsc.parallel_loop(0, _PER_W, step=_LANES)
        def _body(i):
            off = pl.multiple_of(i, _LANES)
            x = step_v[pl.ds(off, _LANES)]
            r = (x * float(_TIMESTEPS) + _MAGIC) - _MAGIC
            idx = r.astype(jnp.int32)
            out_v[pl.ds(off, _LANES)] = plsc.load_gather(gamma_v, [idx])
        pltpu.sync_copy(out_v, out_hbm.at[pl.ds(base, _PER_W)])

    return _noise_lookup


_kernel_fn = _make_kernel()


def kernel(step, gamma):
    return _kernel_fn(step, gamma)
